# baseline (speedup 1.0000x reference)
import functools

import jax
import jax.numpy as jnp
from jax import lax
from jax.experimental import pallas as pl
from jax.experimental.pallas import tpu as pltpu
from jax.experimental.pallas import tpu_sc as plsc

NUM_CORES = 2
NUM_SUBCORES = 16
LANES = 16
NW = NUM_CORES * NUM_SUBCORES

PAIRS = 768
ROWS = PAIRS // NW
DIM = 12
PERIODS = 288
PT_WORDS = PERIODS * DIM
WT_OFF = 2 * PAIRS
WT_WORDS = 7 * DIM

NODES = 10000
NROWS = 416
NTAIL = NODES - 23 * NROWS


def _sc_body(xswt, tef, ptf, nodes, node_out, out, xbuf, obuf, tebuf, ptbuf,
             wtbuf, outbuf, nbuf, sem, nsem):
    wid = lax.axis_index("s") * NUM_CORES + lax.axis_index("c")
    base = wid * ROWS

    @pl.when(wid < 24)
    def _():
        pltpu.async_copy(nodes.at[pl.ds(wid * NROWS, NROWS), :], nbuf,
                         nsem).wait()
        pltpu.async_copy(nbuf, node_out.at[pl.ds(wid * NROWS, NROWS), :],
                         nsem).wait()

    @pl.when(wid == 24)
    def _():
        pltpu.async_copy(nodes.at[pl.ds(24 * NROWS, NODES - 24 * NROWS), :],
                         nbuf.at[pl.ds(0, NODES - 24 * NROWS), :], nsem).wait()
        pltpu.async_copy(nbuf.at[pl.ds(0, NODES - 24 * NROWS), :],
                         node_out.at[pl.ds(24 * NROWS, NODES - 24 * NROWS), :],
                         nsem).wait()

    c1 = pltpu.async_copy(xswt.at[pl.ds(2 * base, 2 * ROWS)],
                          xbuf.at[pl.ds(0, 2 * ROWS)], sem)
    c2 = pltpu.async_copy(tef.at[pl.ds(base * DIM, ROWS * DIM)],
                          tebuf.at[pl.ds(0, ROWS * DIM)], sem)
    c3 = pltpu.async_copy(ptf, ptbuf.at[pl.ds(0, PT_WORDS)], sem)
    c4 = pltpu.async_copy(xswt.at[pl.ds(WT_OFF, WT_WORDS)],
                          wtbuf.at[pl.ds(0, WT_WORDS)], sem)
    c1.wait()

    par = lax.iota(jnp.int32, LANES) % 2
    scale = jnp.where(par == 0, float(PERIODS), 1.0)
    hi = jnp.where(par == 0, PERIODS - 1, 6)
    for k in range(2 * ROWS // LANES):
        v = xbuf[pl.ds(k * LANES, LANES)] * scale
        i = v.astype(jnp.int32)
        i = jnp.where(i.astype(jnp.float32) > v, i - 1, i)
        obuf[pl.ds(k * LANES, LANES)] = jnp.clip(i, 0, hi) * DIM

    c2.wait()
    c3.wait()
    c4.wait()

    def body(r, carry):
        p12 = obuf[pl.ds(2 * r, LANES)][0]
        w12 = obuf[pl.ds(2 * r + 1, LANES)][0]
        tev = tebuf[pl.ds(r * DIM, LANES)]
        pe = ptbuf[pl.ds(p12, LANES)]
        we = wtbuf[pl.ds(w12, LANES)]
        outbuf[pl.ds(r * DIM, LANES)] = tev * pe * we
        return carry

    lax.fori_loop(0, ROWS, body, 0)

    pltpu.sync_copy(outbuf.at[pl.ds(0, ROWS * DIM)],
                    out.at[pl.ds(base * DIM, ROWS * DIM)])


_sc_encoder = functools.partial(
    pl.kernel,
    mesh=plsc.VectorSubcoreMesh(core_axis_name="c", subcore_axis_name="s"),
    out_type=(
        jax.ShapeDtypeStruct((NODES, DIM), jnp.float32),
        jax.ShapeDtypeStruct((PAIRS * DIM,), jnp.float32),
    ),
    scratch_types=[
        pltpu.VMEM((2 * ROWS + LANES,), jnp.float32),
        pltpu.VMEM((2 * ROWS + LANES,), jnp.int32),
        pltpu.VMEM((ROWS * DIM + LANES,), jnp.float32),
        pltpu.VMEM((PT_WORDS + LANES,), jnp.float32),
        pltpu.VMEM((WT_WORDS + LANES,), jnp.float32),
        pltpu.VMEM((ROWS * DIM + LANES,), jnp.float32),
        pltpu.VMEM((NROWS, DIM), jnp.float32),
        pltpu.SemaphoreType.DMA,
        pltpu.SemaphoreType.DMA,
    ],
)(_sc_body)


def kernel(x, periods_table, weekend_table, node_embeddings, time_embeddings):
    b, t = x.shape[0], x.shape[1]
    xswt = jnp.concatenate([
        x[:, :, 0, 1:3].reshape(2 * b * t),
        weekend_table.reshape(WT_WORDS),
    ])
    tef = time_embeddings[:b].reshape(b * t * DIM)
    ptf = periods_table.reshape(PT_WORDS)
    node_out, out = _sc_encoder(xswt, tef, ptf, node_embeddings)
    return node_out, out.reshape(b, t, DIM)


# speedup vs baseline: 1.2273x; 1.2273x over previous
import functools

import jax
import jax.numpy as jnp
from jax import lax
from jax.experimental import pallas as pl
from jax.experimental.pallas import tpu as pltpu
from jax.experimental.pallas import tpu_sc as plsc

NUM_CORES = 2
NUM_SUBCORES = 16
LANES = 16
NW = NUM_CORES * NUM_SUBCORES

PAIRS = 768
ROWS = PAIRS // NW
DIM = 12
PERIODS = 288
PT_WORDS = PERIODS * DIM
WT_WORDS = 7 * DIM


def _sc_body(xsw, tef, ptf, wtf, out, xbuf, obuf, tebuf, ptbuf, wtbuf,
             outbuf, sem):
    wid = lax.axis_index("s") * NUM_CORES + lax.axis_index("c")
    base = wid * ROWS

    c1 = pltpu.async_copy(xsw.at[pl.ds(2 * base, 2 * ROWS)],
                          xbuf.at[pl.ds(0, 2 * ROWS)], sem)
    c2 = pltpu.async_copy(tef.at[pl.ds(base * DIM, ROWS * DIM)],
                          tebuf.at[pl.ds(0, ROWS * DIM)], sem)
    c3 = pltpu.async_copy(ptf, ptbuf.at[pl.ds(0, PT_WORDS)], sem)
    c4 = pltpu.async_copy(wtf, wtbuf.at[pl.ds(0, WT_WORDS)], sem)
    c1.wait()

    par = lax.iota(jnp.int32, LANES) % 2
    scale = jnp.where(par == 0, float(PERIODS), 1.0)
    hi = jnp.where(par == 0, PERIODS - 1, 6)
    for k in range(2 * ROWS // LANES):
        v = xbuf[pl.ds(k * LANES, LANES)] * scale
        i = v.astype(jnp.int32)
        i = jnp.where(i.astype(jnp.float32) > v, i - 1, i)
        obuf[pl.ds(k * LANES, LANES)] = jnp.clip(i, 0, hi) * DIM

    c2.wait()
    c3.wait()
    c4.wait()

    def body(r, carry):
        p12 = obuf[pl.ds(2 * r, LANES)][0]
        w12 = obuf[pl.ds(2 * r + 1, LANES)][0]
        tev = tebuf[pl.ds(r * DIM, LANES)]
        pe = ptbuf[pl.ds(p12, LANES)]
        we = wtbuf[pl.ds(w12, LANES)]
        outbuf[pl.ds(r * DIM, LANES)] = tev * pe * we
        return carry

    lax.fori_loop(0, ROWS, body, 0)

    pltpu.sync_copy(outbuf.at[pl.ds(0, ROWS * DIM)],
                    out.at[pl.ds(base * DIM, ROWS * DIM)])


_sc_encoder = functools.partial(
    pl.kernel,
    mesh=plsc.VectorSubcoreMesh(core_axis_name="c", subcore_axis_name="s"),
    out_type=jax.ShapeDtypeStruct((PAIRS * DIM,), jnp.float32),
    scratch_types=[
        pltpu.VMEM((2 * ROWS + LANES,), jnp.float32),
        pltpu.VMEM((2 * ROWS + LANES,), jnp.int32),
        pltpu.VMEM((ROWS * DIM + LANES,), jnp.float32),
        pltpu.VMEM((PT_WORDS + LANES,), jnp.float32),
        pltpu.VMEM((WT_WORDS + LANES,), jnp.float32),
        pltpu.VMEM((ROWS * DIM + LANES,), jnp.float32),
        pltpu.SemaphoreType.DMA,
    ],
)(_sc_body)


def kernel(x, periods_table, weekend_table, node_embeddings, time_embeddings):
    b, t = x.shape[0], x.shape[1]
    xsw = x[:, :, 0, 1:3].reshape(2 * b * t)
    tef = time_embeddings[:b].reshape(b * t * DIM)
    ptf = periods_table.reshape(PT_WORDS)
    wtf = weekend_table.reshape(WT_WORDS)
    out = _sc_encoder(xsw, tef, ptf, wtf)
    return node_embeddings, out.reshape(b, t, DIM)


# speedup vs baseline: 1.4779x; 1.2042x over previous
import functools

import jax
import jax.numpy as jnp
from jax import lax
from jax.experimental import pallas as pl
from jax.experimental.pallas import tpu as pltpu
from jax.experimental.pallas import tpu_sc as plsc

NUM_CORES = 2
NUM_SUBCORES = 16
LANES = 16
NW = NUM_CORES * NUM_SUBCORES

PAIRS = 768
ROWS = PAIRS // NW
DIM = 12
PERIODS = 288
PT_WORDS = PERIODS * DIM
WT_WORDS = 7 * DIM


def _floor_off(v, hi):
    i = v.astype(jnp.int32)
    i = jnp.where(i.astype(jnp.float32) > v, i - 1, i)
    return jnp.clip(i, 0, hi) * DIM


def _sc_body(xsw, tef, ptf, wtf, out, xbuf, pibuf, wibuf, tebuf, ptbuf,
             wtbuf, outbuf, sem):
    wid = lax.axis_index("s") * NUM_CORES + lax.axis_index("c")
    base = wid * ROWS

    c1 = pltpu.async_copy(xsw.at[pl.ds(base, ROWS)], xbuf.at[pl.ds(0, ROWS)], sem)
    c2 = pltpu.async_copy(xsw.at[pl.ds(PAIRS + base, ROWS)],
                          xbuf.at[pl.ds(32, ROWS)], sem)
    c3 = pltpu.async_copy(tef.at[pl.ds(base * DIM, ROWS * DIM)],
                          tebuf.at[pl.ds(0, ROWS * DIM)], sem)
    c4 = pltpu.async_copy(ptf, ptbuf.at[pl.ds(0, PT_WORDS)], sem)
    c5 = pltpu.async_copy(wtf, wtbuf.at[pl.ds(0, WT_WORDS)], sem)
    c1.wait()
    c2.wait()

    for off in (0, ROWS - LANES):
        pv = xbuf[pl.ds(off, LANES)]
        wv = xbuf[pl.ds(32 + off, LANES)]
        pibuf[pl.ds(off, LANES)] = _floor_off(pv * float(PERIODS), PERIODS - 1)
        wibuf[pl.ds(off, LANES)] = _floor_off(wv, 6)

    c3.wait()
    c4.wait()
    c5.wait()

    def body(r, carry):
        p12 = pibuf[pl.ds(r, LANES)][0]
        w12 = wibuf[pl.ds(r, LANES)][0]
        tev = tebuf[pl.ds(r * DIM, LANES)]
        pe = ptbuf[pl.ds(p12, LANES)]
        we = wtbuf[pl.ds(w12, LANES)]
        outbuf[pl.ds(r * DIM, LANES)] = tev * pe * we
        return carry

    lax.fori_loop(0, ROWS, body, 0)

    pltpu.sync_copy(outbuf.at[pl.ds(0, ROWS * DIM)],
                    out.at[pl.ds(base * DIM, ROWS * DIM)])


_sc_encoder = functools.partial(
    pl.kernel,
    mesh=plsc.VectorSubcoreMesh(core_axis_name="c", subcore_axis_name="s"),
    out_type=jax.ShapeDtypeStruct((PAIRS * DIM,), jnp.float32),
    scratch_types=[
        pltpu.VMEM((64,), jnp.float32),
        pltpu.VMEM((ROWS + LANES,), jnp.int32),
        pltpu.VMEM((ROWS + LANES,), jnp.int32),
        pltpu.VMEM((ROWS * DIM + LANES,), jnp.float32),
        pltpu.VMEM((PT_WORDS + LANES,), jnp.float32),
        pltpu.VMEM((WT_WORDS + LANES,), jnp.float32),
        pltpu.VMEM((ROWS * DIM + LANES,), jnp.float32),
        pltpu.SemaphoreType.DMA,
    ],
)(_sc_body)


def kernel(x, periods_table, weekend_table, node_embeddings, time_embeddings):
    b, t = x.shape[0], x.shape[1]
    xsw = x[:, :, 0, 1:3].transpose(2, 0, 1).reshape(2 * b * t)
    tef = time_embeddings[:b].reshape(b * t * DIM)
    ptf = periods_table.reshape(PT_WORDS)
    wtf = weekend_table.reshape(WT_WORDS)
    out = _sc_encoder(xsw, tef, ptf, wtf)
    return node_embeddings, out.reshape(b, t, DIM)
